# Initial kernel scaffold; baseline (speedup 1.0000x reference)
import jax
import jax.numpy as jnp
from jax.experimental import pallas as pl


def kernel(inp, hsv, w_deconv, b_deconv, wsh, bsh, wgb, bgb, wcv, bcv):
    raise NotImplementedError("write your pallas kernel here")


# speedup vs baseline: 1.3873x; 1.3873x over previous
import functools

import jax
import jax.numpy as jnp
from jax.experimental import pallas as pl
from jax.experimental.pallas import tpu as pltpu

_EPS = 1e-5
_F32 = jnp.float32
_BF16 = jnp.bfloat16


def _vmem_limit():
    cap = 64 * 1024 * 1024
    return int(min((cap * 3) // 4, 100 * 1024 * 1024))


def _zero_halo(ref, dtype):
    hp, wp, c = ref.shape
    ref[0:1, :, :] = jnp.zeros((1, wp, c), dtype)
    ref[hp - 1:hp, :, :] = jnp.zeros((1, wp, c), dtype)
    ref[:, 0:1, :] = jnp.zeros((hp, 1, c), dtype)
    ref[:, wp - 1:wp, :] = jnp.zeros((hp, 1, c), dtype)


def _instance_norm_rows(x2d, eps=_EPS):
    mean = jnp.mean(x2d, axis=0, keepdims=True)
    cen = x2d - mean
    var = jnp.mean(cen * cen, axis=0, keepdims=True)
    return cen * jax.lax.rsqrt(var + eps)


def _im2col3x3(pad_ref, pat_ref, h, w, c):
    for t in range(9):
        u, v = t // 3, t % 3
        pat_ref[:, t * c:(t + 1) * c] = (
            pad_ref[u:u + h, v:v + w, :].reshape(h * w, c).astype(_BF16))


def _fused_kernel(x_ref, seg_ref, wdc_ref, bdc_ref, wsh_ref, bsh_ref,
                  wgb_ref, bgb_ref, wcv_ref, bcv_ref, o_ref,
                  xpad, patd, x0, segpad, pat, actpad0, actpad1, spad, *,
                  H, W, Cin, C, Cs, nh):
    Hx, Wx = 2 * H, 2 * W
    HW = Hx * Wx

    _zero_halo(xpad, _BF16)
    xpad[1:H + 1, 1:W + 1, :] = jnp.maximum(x_ref[0], 0)
    for t in range(9):
        u, v = t // 3, t % 3
        patd[:, t * Cin:(t + 1) * Cin] = (
            xpad[u:u + H, v:v + W, :].reshape(H * W, Cin))
    acc = jnp.dot(patd[...], wdc_ref[...],
                  preferred_element_type=jnp.float32) + bdc_ref[...]

    for ph in range(2):
        for pw in range(2):
            g = 2 * ph + pw
            x0[ph:Hx:2, pw:Wx:2, :] = (
                acc[:, g * C:(g + 1) * C].reshape(H, W, C))

    _zero_halo(segpad, _F32)
    sv = seg_ref[0].astype(_F32)
    for a in range(2):
        for b in range(2):
            segpad[1 + a:Hx + 1:2, 1 + b:Wx + 1:2, :] = sv

    _im2col3x3(segpad, pat, Hx, Wx, Cs)
    mlp = jnp.dot(pat[...], wsh_ref[...],
                  preferred_element_type=jnp.float32) + bsh_ref[...]
    mlp = jnp.maximum(mlp, 0.0)
    _zero_halo(actpad0, _BF16)
    _zero_halo(actpad1, _BF16)
    actpad0[1:Hx + 1, 1:Wx + 1, :] = (
        mlp[:, :nh].astype(_BF16).reshape(Hx, Wx, nh))
    actpad1[1:Hx + 1, 1:Wx + 1, :] = (
        mlp[:, nh:].astype(_BF16).reshape(Hx, Wx, nh))
    _zero_halo(spad, _BF16)

    x0f = x0[...].reshape(HW, C)

    def spade_stage(h2d, apad, st):
        normalized = _instance_norm_rows(h2d)
        _im2col3x3(apad, pat, Hx, Wx, nh)
        gb = jnp.dot(pat[...], wgb_ref[st],
                     preferred_element_type=jnp.float32) + bgb_ref[st]
        s = normalized * (1.0 + gb[:, :C]) + gb[:, C:]
        s = jnp.where(s >= 0.0, s, 0.2 * s)
        spad[1:Hx + 1, 1:Wx + 1, :] = s.astype(_BF16).reshape(Hx, Wx, C)
        _im2col3x3(spad, pat, Hx, Wx, C)
        return (jnp.dot(pat[...], wcv_ref[st],
                        preferred_element_type=jnp.float32) + bcv_ref[st])

    dx = spade_stage(x0f, actpad0, 0)
    dx = spade_stage(dx, actpad1, 1)
    y = _instance_norm_rows(x0f + dx)
    o_ref[0] = y.T


def kernel(inp, hsv, w_deconv, b_deconv, wsh, bsh, wgb, bgb, wcv, bcv):
    N, H, W, Cin = inp.shape
    Cs = hsv.shape[-1]
    nh = wsh.shape[-1]
    C = wcv.shape[-1]
    assert W % 8 == 0 and hsv.shape[1] == H and hsv.shape[2] == W
    Hx, Wx = 2 * H, 2 * W
    HW = Hx * Wx

    x_bf = inp.astype(_BF16)
    seg_bf = hsv.astype(_BF16)
    wdc = w_deconv.astype(_BF16)
    b4 = jnp.tile(b_deconv.astype(_F32), 4).reshape(1, 4 * C)
    wsh_c = jnp.concatenate([wsh[0], wsh[1]], axis=-1).astype(_BF16)
    bsh_c = jnp.concatenate([bsh[0], bsh[1]], axis=-1).astype(_F32)
    wgb_c = wgb.reshape(2, 9 * nh, 2 * C).astype(_BF16)
    wcv_c = wcv.astype(_BF16)
    bgb_c = bgb.astype(_F32)
    bcv_c = bcv.astype(_F32)

    flops = 2 * N * (H * W * 9 * Cin * 4 * C + HW * 9 * Cs * 2 * nh
                     + 2 * HW * (9 * nh * 2 * C + 9 * C * C))
    bytes_accessed = 2 * (N * H * W * (Cin + Cs) + 9 * Cin * 4 * C
                          + 9 * Cs * 2 * nh + 2 * 9 * nh * 2 * C
                          + 2 * 9 * C * C) + 4 * N * HW * C

    out = pl.pallas_call(
        functools.partial(_fused_kernel, H=H, W=W, Cin=Cin, C=C, Cs=Cs,
                          nh=nh),
        out_shape=jax.ShapeDtypeStruct((N, C, HW), _F32),
        grid=(N,),
        in_specs=[
            pl.BlockSpec((1, H, W, Cin), lambda n: (n, 0, 0, 0)),
            pl.BlockSpec((1, H, W, Cs), lambda n: (n, 0, 0, 0)),
            pl.BlockSpec((9 * Cin, 4 * C), lambda n: (0, 0)),
            pl.BlockSpec((1, 4 * C), lambda n: (0, 0)),
            pl.BlockSpec((9 * Cs, 2 * nh), lambda n: (0, 0)),
            pl.BlockSpec((1, 2 * nh), lambda n: (0, 0)),
            pl.BlockSpec((2, 9 * nh, 2 * C), lambda n: (0, 0, 0)),
            pl.BlockSpec((2, 1, 2 * C), lambda n: (0, 0, 0)),
            pl.BlockSpec((2, 9 * C, C), lambda n: (0, 0, 0)),
            pl.BlockSpec((2, 1, C), lambda n: (0, 0, 0)),
        ],
        out_specs=pl.BlockSpec((1, C, HW), lambda n: (n, 0, 0)),
        scratch_shapes=[
            pltpu.VMEM((H + 2, W + 2, Cin), _BF16),
            pltpu.VMEM((H * W, 9 * Cin), _BF16),
            pltpu.VMEM((Hx, Wx, C), _F32),
            pltpu.VMEM((Hx + 2, Wx + 2, Cs), _F32),
            pltpu.VMEM((HW, 9 * Cs), _BF16),
            pltpu.VMEM((Hx + 2, Wx + 2, nh), _BF16),
            pltpu.VMEM((Hx + 2, Wx + 2, nh), _BF16),
            pltpu.VMEM((Hx + 2, Wx + 2, C), _BF16),
        ],
        compiler_params=pltpu.CompilerParams(
            dimension_semantics=("parallel",),
            vmem_limit_bytes=_vmem_limit()),
        cost_estimate=pl.CostEstimate(flops=flops, transcendentals=0,
                                      bytes_accessed=bytes_accessed),
    )(x_bf, seg_bf, wdc, b4, wsh_c, bsh_c, wgb_c, bgb_c, wcv_c, bcv_c)
    return out.reshape(N, C, Hx, Wx)


# speedup vs baseline: 1.5835x; 1.1414x over previous
import functools

import jax
import jax.numpy as jnp
from jax.experimental import pallas as pl
from jax.experimental.pallas import tpu as pltpu

_EPS = 1e-5
_F32 = jnp.float32
_BF16 = jnp.bfloat16

_TAP_PERM = (0, 1, 3, 4, 6, 7, 2, 5, 8)


def _vmem_limit():
    cap = 64 * 1024 * 1024
    return int(min((cap * 3) // 4, 100 * 1024 * 1024))


def _perm_taps(w9c, c):
    n = w9c.shape[-1]
    return w9c.reshape(9, c, n)[jnp.array(_TAP_PERM)].reshape(9 * c, n)


def _zero_halo(ref, dtype):
    hp, wp, c = ref.shape
    ref[0:1, :, :] = jnp.zeros((1, wp, c), dtype)
    ref[hp - 1:hp, :, :] = jnp.zeros((1, wp, c), dtype)
    ref[:, 0:1, :] = jnp.zeros((hp, 1, c), dtype)
    ref[:, wp - 1:wp, :] = jnp.zeros((hp, 1, c), dtype)


def _instance_norm_rows(x2d, eps=_EPS):
    mean = jnp.mean(x2d, axis=0, keepdims=True)
    msq = jnp.mean(x2d * x2d, axis=0, keepdims=True)
    var = msq - mean * mean
    return (x2d - mean) * jax.lax.rsqrt(var + eps)


def _norm_scale_bias(x2d, eps=_EPS):
    mean = jnp.mean(x2d, axis=0, keepdims=True)
    msq = jnp.mean(x2d * x2d, axis=0, keepdims=True)
    inv = jax.lax.rsqrt(msq - mean * mean + eps)
    return inv, -mean * inv


def _conv_shift(pad_ref, csA, csB, h, w, c):
    csA[:, :, 0:c] = pad_ref[:, 0:w, :].astype(_BF16)
    csA[:, :, c:2 * c] = pad_ref[:, 1:w + 1, :].astype(_BF16)
    csB[:, :, 0:c] = pad_ref[:, 2:w + 2, :].astype(_BF16)
    csB[0:h + 1, :, c:2 * c] = csB[1:h + 2, :, 0:c]


def _conv_dots(csA, csB, w_ref, h, w, c):
    a = csA[...].reshape((h + 2) * w, 2 * c)
    bb = csB[...].reshape((h + 2) * w, 2 * c)
    hw = h * w
    dot = functools.partial(jnp.dot, preferred_element_type=jnp.float32)
    acc = dot(a[0:hw], w_ref[0:2 * c])
    acc = acc + dot(a[w:w + hw], w_ref[2 * c:4 * c])
    acc = acc + dot(a[2 * w:2 * w + hw], w_ref[4 * c:6 * c])
    acc = acc + dot(bb[0:hw], w_ref[6 * c:8 * c])
    acc = acc + dot(bb[2 * w:2 * w + hw, 0:c], w_ref[8 * c:9 * c])
    return acc


def _sample_stages(i, x_ref, seg_ref, wdc_ref, bdc_ref, wsh_ref, bsh_ref,
                   wgb_ref, bgb_ref, wcv_ref, bcv_ref, o_ref, scr,
                   H, W, Cin, C, Cs, nh):
    (xpad, csdA, csdB, x0, segpad, cssA, cssB, csgA, csgB, cscA, cscB,
     actpad0, actpad1, spad0, spad1) = scr
    Hx, Wx = 2 * H, 2 * W
    HW = Hx * Wx

    _zero_halo(xpad, _BF16)
    xpad[1:H + 1, 1:W + 1, :] = jnp.maximum(x_ref[i], 0)
    _conv_shift(xpad, csdA, csdB, H, W, Cin)
    yield
    acc = _conv_dots(csdA, csdB, wdc_ref, H, W, Cin) + bdc_ref[...]
    yield
    for ph in range(2):
        x0[ph:Hx:2, :, :] = acc[:, 2 * ph * C:2 * (ph + 1) * C].reshape(
            H, Wx, C)
    yield
    _zero_halo(segpad, _F32)
    sv = seg_ref[i].astype(_F32)
    for a in range(2):
        for b in range(2):
            segpad[1 + a:Hx + 1:2, 1 + b:Wx + 1:2, :] = sv
    yield
    _conv_shift(segpad, cssA, cssB, Hx, Wx, Cs)
    yield
    mlp = _conv_dots(cssA, cssB, wsh_ref, Hx, Wx, Cs) + bsh_ref[...]
    mlp = jnp.maximum(mlp, 0.0)
    yield
    _zero_halo(actpad0, _BF16)
    _zero_halo(actpad1, _BF16)
    actpad0[1:Hx + 1, 1:Wx + 1, :] = (
        mlp[:, :nh].astype(_BF16).reshape(Hx, Wx, nh))
    actpad1[1:Hx + 1, 1:Wx + 1, :] = (
        mlp[:, nh:].astype(_BF16).reshape(Hx, Wx, nh))
    _zero_halo(spad0, _BF16)
    _zero_halo(spad1, _BF16)
    yield
    x0f = x0[...].reshape(HW, C)

    def modulate(h2d, gb):
        inv, nbias = _norm_scale_bias(h2d)
        gbb = gb.astype(_BF16)
        nrm = (h2d.astype(_BF16) * inv.astype(_BF16)[0]
               + nbias.astype(_BF16)[0])
        s = nrm * (_BF16(1.0) + gbb[:, :C]) + gbb[:, C:]
        return jnp.where(s >= 0, s, _BF16(0.2) * s)

    _conv_shift(actpad0, csgA, csgB, Hx, Wx, nh)
    yield
    gb = _conv_dots(csgA, csgB, wgb_ref[0], Hx, Wx, nh) + bgb_ref[0]
    yield
    spad0[1:Hx + 1, 1:Wx + 1, :] = modulate(x0f, gb).reshape(Hx, Wx, C)
    yield
    _conv_shift(spad0, cscA, cscB, Hx, Wx, C)
    yield
    dx = _conv_dots(cscA, cscB, wcv_ref[0], Hx, Wx, C) + bcv_ref[0]
    yield
    _conv_shift(actpad1, csgA, csgB, Hx, Wx, nh)
    yield
    gb = _conv_dots(csgA, csgB, wgb_ref[1], Hx, Wx, nh) + bgb_ref[1]
    yield
    spad1[1:Hx + 1, 1:Wx + 1, :] = modulate(dx, gb).reshape(Hx, Wx, C)
    yield
    _conv_shift(spad1, cscA, cscB, Hx, Wx, C)
    yield
    dx = _conv_dots(cscA, cscB, wcv_ref[1], Hx, Wx, C) + bcv_ref[1]
    yield
    y = _instance_norm_rows(x0f + dx)
    yield
    o_ref[i] = y.T


def _fused_kernel(x_ref, seg_ref, wdc_ref, bdc_ref, wsh_ref, bsh_ref,
                  wgb_ref, bgb_ref, wcv_ref, bcv_ref, o_ref, *scr,
                  H, W, Cin, C, Cs, nh):
    half = len(scr) // 2
    gens = [_sample_stages(i, x_ref, seg_ref, wdc_ref, bdc_ref, wsh_ref,
                           bsh_ref, wgb_ref, bgb_ref, wcv_ref, bcv_ref,
                           o_ref, scr[i * half:(i + 1) * half],
                           H, W, Cin, C, Cs, nh)
            for i in range(2)]
    live = [True, True]
    for _ in range(3):
        next(gens[0], None)
    while any(live):
        for i in range(2):
            if live[i] and next(gens[i], "done") == "done":
                live[i] = False


def kernel(inp, hsv, w_deconv, b_deconv, wsh, bsh, wgb, bgb, wcv, bcv):
    N, H, W, Cin = inp.shape
    Cs = hsv.shape[-1]
    nh = wsh.shape[-1]
    C = wcv.shape[-1]
    assert W % 8 == 0 and hsv.shape[1] == H and hsv.shape[2] == W
    assert Cs == nh == C and N % 2 == 0
    Hx, Wx = 2 * H, 2 * W
    HW = Hx * Wx

    x_bf = inp.astype(_BF16)
    seg_bf = hsv.astype(_BF16)
    wdc = _perm_taps(w_deconv, Cin).astype(_BF16)
    b4 = jnp.tile(b_deconv.astype(_F32), 4).reshape(1, 4 * C)
    wsh_c = _perm_taps(jnp.concatenate([wsh[0], wsh[1]], axis=-1),
                       Cs).astype(_BF16)
    bsh_c = jnp.concatenate([bsh[0], bsh[1]], axis=-1).astype(_F32)
    wgb_c = jnp.stack([_perm_taps(wgb.reshape(2, 9 * nh, 2 * C)[i], nh)
                       for i in range(2)]).astype(_BF16)
    wcv_c = jnp.stack([_perm_taps(wcv[i], C)
                       for i in range(2)]).astype(_BF16)
    bgb_c = bgb.astype(_F32)
    bcv_c = bcv.astype(_F32)

    flops = 2 * N * (H * W * 9 * Cin * 4 * C + HW * 9 * Cs * 2 * nh
                     + 2 * HW * (9 * nh * 2 * C + 9 * C * C))
    bytes_accessed = 2 * (N * H * W * (Cin + Cs) + 9 * Cin * 4 * C
                          + 9 * Cs * 2 * nh + 2 * 9 * nh * 2 * C
                          + 2 * 9 * C * C) + 4 * N * HW * C

    out = pl.pallas_call(
        functools.partial(_fused_kernel, H=H, W=W, Cin=Cin, C=C, Cs=Cs,
                          nh=nh),
        out_shape=jax.ShapeDtypeStruct((N, C, HW), _F32),
        grid=(N // 2,),
        in_specs=[
            pl.BlockSpec((2, H, W, Cin), lambda n: (n, 0, 0, 0)),
            pl.BlockSpec((2, H, W, Cs), lambda n: (n, 0, 0, 0)),
            pl.BlockSpec((9 * Cin, 4 * C), lambda n: (0, 0)),
            pl.BlockSpec((1, 4 * C), lambda n: (0, 0)),
            pl.BlockSpec((9 * Cs, 2 * nh), lambda n: (0, 0)),
            pl.BlockSpec((1, 2 * nh), lambda n: (0, 0)),
            pl.BlockSpec((2, 9 * nh, 2 * C), lambda n: (0, 0, 0)),
            pl.BlockSpec((2, 1, 2 * C), lambda n: (0, 0, 0)),
            pl.BlockSpec((2, 9 * C, C), lambda n: (0, 0, 0)),
            pl.BlockSpec((2, 1, C), lambda n: (0, 0, 0)),
        ],
        out_specs=pl.BlockSpec((2, C, HW), lambda n: (n, 0, 0)),
        scratch_shapes=[
            s for _ in range(2) for s in [
                pltpu.VMEM((H + 2, W + 2, Cin), _BF16),
                pltpu.VMEM((H + 2, W, 2 * Cin), _BF16),
                pltpu.VMEM((H + 2, W, 2 * Cin), _BF16),
                pltpu.VMEM((Hx, Wx, C), _F32),
                pltpu.VMEM((Hx + 2, Wx + 2, Cs), _F32),
                pltpu.VMEM((Hx + 2, Wx, 2 * C), _BF16),
                pltpu.VMEM((Hx + 2, Wx, 2 * C), _BF16),
                pltpu.VMEM((Hx + 2, Wx, 2 * C), _BF16),
                pltpu.VMEM((Hx + 2, Wx, 2 * C), _BF16),
                pltpu.VMEM((Hx + 2, Wx, 2 * C), _BF16),
                pltpu.VMEM((Hx + 2, Wx, 2 * C), _BF16),
                pltpu.VMEM((Hx + 2, Wx + 2, nh), _BF16),
                pltpu.VMEM((Hx + 2, Wx + 2, nh), _BF16),
                pltpu.VMEM((Hx + 2, Wx + 2, C), _BF16),
                pltpu.VMEM((Hx + 2, Wx + 2, C), _BF16),
            ]],
        compiler_params=pltpu.CompilerParams(
            dimension_semantics=("parallel",),
            vmem_limit_bytes=_vmem_limit()),
        cost_estimate=pl.CostEstimate(flops=flops, transcendentals=0,
                                      bytes_accessed=bytes_accessed),
    )(x_bf, seg_bf, wdc, b4, wsh_c, bsh_c, wgb_c, bgb_c, wcv_c, bcv_c)
    return out.reshape(N, C, Hx, Wx)


# speedup vs baseline: 1.7210x; 1.0868x over previous
import functools

import jax
import jax.numpy as jnp
from jax.experimental import pallas as pl
from jax.experimental.pallas import tpu as pltpu

_EPS = 1e-5
_F32 = jnp.float32
_BF16 = jnp.bfloat16

_TAP_PERM = (0, 1, 3, 4, 6, 7, 2, 5, 8)


def _vmem_limit():
    cap = 64 * 1024 * 1024
    return int(min((cap * 3) // 4, 100 * 1024 * 1024))


def _perm_taps(w9c, c):
    n = w9c.shape[-1]
    return w9c.reshape(9, c, n)[jnp.array(_TAP_PERM)].reshape(9 * c, n)


def _zero_halo(ref, dtype):
    hp, wp, c = ref.shape
    ref[0:1, :, :] = jnp.zeros((1, wp, c), dtype)
    ref[hp - 1:hp, :, :] = jnp.zeros((1, wp, c), dtype)
    ref[:, 0:1, :] = jnp.zeros((hp, 1, c), dtype)
    ref[:, wp - 1:wp, :] = jnp.zeros((hp, 1, c), dtype)


def _instance_norm_rows(x2d, eps=_EPS):
    mean = jnp.mean(x2d, axis=0, keepdims=True)
    msq = jnp.mean(x2d * x2d, axis=0, keepdims=True)
    var = msq - mean * mean
    return (x2d - mean) * jax.lax.rsqrt(var + eps)


def _norm_scale_bias(x2d, eps=_EPS):
    mean = jnp.mean(x2d, axis=0, keepdims=True)
    msq = jnp.mean(x2d * x2d, axis=0, keepdims=True)
    inv = jax.lax.rsqrt(msq - mean * mean + eps)
    return inv, -mean * inv


def _conv_shift(pad_ref, csA, csB, h, w, c):
    csA[:, :, 0:c] = pad_ref[:, 0:w, :].astype(_BF16)
    csA[:, :, c:2 * c] = pad_ref[:, 1:w + 1, :].astype(_BF16)
    csB[:, :, 0:c] = pad_ref[:, 2:w + 2, :].astype(_BF16)
    csB[0:h + 1, :, c:2 * c] = csB[1:h + 2, :, 0:c]


def _conv_dots(csA, csB, w_ref, h, w, c):
    a = csA[...].reshape((h + 2) * w, 2 * c)
    bb = csB[...].reshape((h + 2) * w, 2 * c)
    hw = h * w
    dot = functools.partial(jnp.dot, preferred_element_type=jnp.float32)
    acc = dot(a[0:hw], w_ref[0:2 * c])
    acc = acc + dot(a[w:w + hw], w_ref[2 * c:4 * c])
    acc = acc + dot(a[2 * w:2 * w + hw], w_ref[4 * c:6 * c])
    acc = acc + dot(bb[0:hw], w_ref[6 * c:8 * c])
    acc = acc + dot(bb[2 * w:2 * w + hw, 0:c], w_ref[8 * c:9 * c])
    return acc


def _sample_stages(i, x_ref, seg_ref, wdc_ref, bdc_ref, wsh_ref, bsh_ref,
                   wgb_ref, bgb_ref, wcv_ref, bcv_ref, o_ref, scr,
                   H, W, Cin, C, Cs, nh):
    (xpad, csdA, csdB, x0, segpad, cssA, cssB, csgA, csgB, cscA, cscB,
     actpad0, actpad1, spad0, spad1) = scr
    Hx, Wx = 2 * H, 2 * W
    HW = Hx * Wx

    _zero_halo(xpad, _BF16)
    xpad[1:H + 1, 1:W + 1, :] = jnp.maximum(x_ref[i], 0)
    _conv_shift(xpad, csdA, csdB, H, W, Cin)
    yield
    acc = _conv_dots(csdA, csdB, wdc_ref, H, W, Cin) + bdc_ref[...]
    yield
    for ph in range(2):
        x0[ph:Hx:2, :, :] = acc[:, 2 * ph * C:2 * (ph + 1) * C].reshape(
            H, Wx, C)
    yield
    _zero_halo(segpad, _F32)
    sv = seg_ref[i].astype(_F32)
    for a in range(2):
        for b in range(2):
            segpad[1 + a:Hx + 1:2, 1 + b:Wx + 1:2, :] = sv
    yield
    _conv_shift(segpad, cssA, cssB, Hx, Wx, Cs)
    yield
    mlp = _conv_dots(cssA, cssB, wsh_ref, Hx, Wx, Cs) + bsh_ref[...]
    mlp = jnp.maximum(mlp, 0.0)
    yield
    _zero_halo(actpad0, _BF16)
    _zero_halo(actpad1, _BF16)
    actpad0[1:Hx + 1, 1:Wx + 1, :] = (
        mlp[:, :nh].astype(_BF16).reshape(Hx, Wx, nh))
    actpad1[1:Hx + 1, 1:Wx + 1, :] = (
        mlp[:, nh:].astype(_BF16).reshape(Hx, Wx, nh))
    _zero_halo(spad0, _BF16)
    _zero_halo(spad1, _BF16)
    yield
    x0f = x0[...].reshape(HW, C)

    def modulate(h2d, gb):
        inv, nbias = _norm_scale_bias(h2d)
        nrm = h2d * inv[0] + nbias[0]
        s = nrm * (1.0 + gb[:, :C]) + gb[:, C:]
        s = jnp.where(s >= 0, s, 0.2 * s)
        return s.astype(_BF16)

    _conv_shift(actpad0, csgA, csgB, Hx, Wx, nh)
    yield
    gb = _conv_dots(csgA, csgB, wgb_ref[0], Hx, Wx, nh) + bgb_ref[0]
    yield
    spad0[1:Hx + 1, 1:Wx + 1, :] = modulate(x0f, gb).reshape(Hx, Wx, C)
    yield
    _conv_shift(spad0, cscA, cscB, Hx, Wx, C)
    yield
    dx = _conv_dots(cscA, cscB, wcv_ref[0], Hx, Wx, C) + bcv_ref[0]
    yield
    _conv_shift(actpad1, csgA, csgB, Hx, Wx, nh)
    yield
    gb = _conv_dots(csgA, csgB, wgb_ref[1], Hx, Wx, nh) + bgb_ref[1]
    yield
    spad1[1:Hx + 1, 1:Wx + 1, :] = modulate(dx, gb).reshape(Hx, Wx, C)
    yield
    _conv_shift(spad1, cscA, cscB, Hx, Wx, C)
    yield
    dx = _conv_dots(cscA, cscB, wcv_ref[1], Hx, Wx, C) + bcv_ref[1]
    yield
    y = _instance_norm_rows(x0f + dx)
    yield
    o_ref[i] = y.T


def _fused_kernel(x_ref, seg_ref, wdc_ref, bdc_ref, wsh_ref, bsh_ref,
                  wgb_ref, bgb_ref, wcv_ref, bcv_ref, o_ref, *scr,
                  H, W, Cin, C, Cs, nh):
    half = len(scr) // 2
    gens = [_sample_stages(i, x_ref, seg_ref, wdc_ref, bdc_ref, wsh_ref,
                           bsh_ref, wgb_ref, bgb_ref, wcv_ref, bcv_ref,
                           o_ref, scr[i * half:(i + 1) * half],
                           H, W, Cin, C, Cs, nh)
            for i in range(2)]
    live = [True, True]
    for _ in range(5):
        next(gens[0], None)
    while any(live):
        for i in range(2):
            if live[i] and next(gens[i], "done") == "done":
                live[i] = False


def kernel(inp, hsv, w_deconv, b_deconv, wsh, bsh, wgb, bgb, wcv, bcv):
    N, H, W, Cin = inp.shape
    Cs = hsv.shape[-1]
    nh = wsh.shape[-1]
    C = wcv.shape[-1]
    assert W % 8 == 0 and hsv.shape[1] == H and hsv.shape[2] == W
    assert Cs == nh == C and N % 2 == 0
    Hx, Wx = 2 * H, 2 * W
    HW = Hx * Wx

    x_bf = inp.astype(_BF16)
    seg_bf = hsv.astype(_BF16)
    wdc = _perm_taps(w_deconv, Cin).astype(_BF16)
    b4 = jnp.tile(b_deconv.astype(_F32), 4).reshape(1, 4 * C)
    wsh_c = _perm_taps(jnp.concatenate([wsh[0], wsh[1]], axis=-1),
                       Cs).astype(_BF16)
    bsh_c = jnp.concatenate([bsh[0], bsh[1]], axis=-1).astype(_F32)
    wgb_c = jnp.stack([_perm_taps(wgb.reshape(2, 9 * nh, 2 * C)[i], nh)
                       for i in range(2)]).astype(_BF16)
    wcv_c = jnp.stack([_perm_taps(wcv[i], C)
                       for i in range(2)]).astype(_BF16)
    bgb_c = bgb.astype(_F32)
    bcv_c = bcv.astype(_F32)

    flops = 2 * N * (H * W * 9 * Cin * 4 * C + HW * 9 * Cs * 2 * nh
                     + 2 * HW * (9 * nh * 2 * C + 9 * C * C))
    bytes_accessed = 2 * (N * H * W * (Cin + Cs) + 9 * Cin * 4 * C
                          + 9 * Cs * 2 * nh + 2 * 9 * nh * 2 * C
                          + 2 * 9 * C * C) + 4 * N * HW * C

    out = pl.pallas_call(
        functools.partial(_fused_kernel, H=H, W=W, Cin=Cin, C=C, Cs=Cs,
                          nh=nh),
        out_shape=jax.ShapeDtypeStruct((N, C, HW), _F32),
        grid=(N // 2,),
        in_specs=[
            pl.BlockSpec((2, H, W, Cin), lambda n: (n, 0, 0, 0)),
            pl.BlockSpec((2, H, W, Cs), lambda n: (n, 0, 0, 0)),
            pl.BlockSpec((9 * Cin, 4 * C), lambda n: (0, 0)),
            pl.BlockSpec((1, 4 * C), lambda n: (0, 0)),
            pl.BlockSpec((9 * Cs, 2 * nh), lambda n: (0, 0)),
            pl.BlockSpec((1, 2 * nh), lambda n: (0, 0)),
            pl.BlockSpec((2, 9 * nh, 2 * C), lambda n: (0, 0, 0)),
            pl.BlockSpec((2, 1, 2 * C), lambda n: (0, 0, 0)),
            pl.BlockSpec((2, 9 * C, C), lambda n: (0, 0, 0)),
            pl.BlockSpec((2, 1, C), lambda n: (0, 0, 0)),
        ],
        out_specs=pl.BlockSpec((2, C, HW), lambda n: (n, 0, 0)),
        scratch_shapes=[
            s for _ in range(2) for s in [
                pltpu.VMEM((H + 2, W + 2, Cin), _BF16),
                pltpu.VMEM((H + 2, W, 2 * Cin), _BF16),
                pltpu.VMEM((H + 2, W, 2 * Cin), _BF16),
                pltpu.VMEM((Hx, Wx, C), _F32),
                pltpu.VMEM((Hx + 2, Wx + 2, Cs), _F32),
                pltpu.VMEM((Hx + 2, Wx, 2 * C), _BF16),
                pltpu.VMEM((Hx + 2, Wx, 2 * C), _BF16),
                pltpu.VMEM((Hx + 2, Wx, 2 * C), _BF16),
                pltpu.VMEM((Hx + 2, Wx, 2 * C), _BF16),
                pltpu.VMEM((Hx + 2, Wx, 2 * C), _BF16),
                pltpu.VMEM((Hx + 2, Wx, 2 * C), _BF16),
                pltpu.VMEM((Hx + 2, Wx + 2, nh), _BF16),
                pltpu.VMEM((Hx + 2, Wx + 2, nh), _BF16),
                pltpu.VMEM((Hx + 2, Wx + 2, C), _BF16),
                pltpu.VMEM((Hx + 2, Wx + 2, C), _BF16),
            ]],
        compiler_params=pltpu.CompilerParams(
            dimension_semantics=("parallel",),
            vmem_limit_bytes=_vmem_limit()),
        cost_estimate=pl.CostEstimate(flops=flops, transcendentals=0,
                                      bytes_accessed=bytes_accessed),
    )(x_bf, seg_bf, wdc, b4, wsh_c, bsh_c, wgb_c, bgb_c, wcv_c, bcv_c)
    return out.reshape(N, C, Hx, Wx)


# speedup vs baseline: 1.8797x; 1.0922x over previous
import functools

import jax
import jax.numpy as jnp
from jax.experimental import pallas as pl
from jax.experimental.pallas import tpu as pltpu

_EPS = 1e-5
_F32 = jnp.float32
_BF16 = jnp.bfloat16

_TAP_PERM = (0, 1, 3, 4, 6, 7, 2, 5, 8)


def _vmem_limit():
    cap = 64 * 1024 * 1024
    return int(min((cap * 3) // 4, 100 * 1024 * 1024))


def _perm_taps(w9c, c):
    n = w9c.shape[-1]
    return w9c.reshape(9, c, n)[jnp.array(_TAP_PERM)].reshape(9 * c, n)


def _zero_halo(ref, dtype):
    hp, wp, c = ref.shape
    ref[0:1, :, :] = jnp.zeros((1, wp, c), dtype)
    ref[hp - 1:hp, :, :] = jnp.zeros((1, wp, c), dtype)
    ref[:, 0:1, :] = jnp.zeros((hp, 1, c), dtype)
    ref[:, wp - 1:wp, :] = jnp.zeros((hp, 1, c), dtype)


def _instance_norm_rows(x2d, eps=_EPS):
    mean = jnp.mean(x2d, axis=0, keepdims=True)
    msq = jnp.mean(x2d * x2d, axis=0, keepdims=True)
    var = msq - mean * mean
    return (x2d - mean) * jax.lax.rsqrt(var + eps)


def _norm_scale_bias(x2d, eps=_EPS):
    mean = jnp.mean(x2d, axis=0, keepdims=True)
    msq = jnp.mean(x2d * x2d, axis=0, keepdims=True)
    inv = jax.lax.rsqrt(msq - mean * mean + eps)
    return inv, -mean * inv


def _conv_shift(pad_ref, csA, csB, h, w, c):
    csA[:, :, 0:c] = pad_ref[:, 0:w, :].astype(_BF16)
    csA[:, :, c:2 * c] = pad_ref[:, 1:w + 1, :].astype(_BF16)
    csB[:, :, 0:c] = pad_ref[:, 2:w + 2, :].astype(_BF16)
    csB[0:h + 1, :, c:2 * c] = csB[1:h + 2, :, 0:c]


def _conv_dots(csA, csB, w_ref, h, w, c):
    a = csA[...].reshape((h + 2) * w, 2 * c)
    bb = csB[...].reshape((h + 2) * w, 2 * c)
    hw = h * w
    dot = functools.partial(jnp.dot, preferred_element_type=jnp.float32)
    acc = dot(a[0:hw], w_ref[0:2 * c])
    acc = acc + dot(a[w:w + hw], w_ref[2 * c:4 * c])
    acc = acc + dot(a[2 * w:2 * w + hw], w_ref[4 * c:6 * c])
    acc = acc + dot(bb[0:hw], w_ref[6 * c:8 * c])
    acc = acc + dot(bb[2 * w:2 * w + hw, 0:c], w_ref[8 * c:9 * c])
    return acc


def _sample_stages(i, x_ref, seg_ref, wdc_ref, bdc_ref, wsh_ref, bsh_ref,
                   wgb_ref, bgb_ref, wcv_ref, bcv_ref, o_ref, scr,
                   H, W, Cin, C, Cs, nh):
    (xpad, csdA, csdB, x0, segpad, cssA, cssB, csgA, csgB, cscA, cscB,
     actpad0, actpad1, spad0, spad1) = scr
    Hx, Wx = 2 * H, 2 * W
    HW = Hx * Wx

    _zero_halo(xpad, _BF16)
    xpad[1:H + 1, 1:W + 1, :] = jnp.maximum(x_ref[i], 0)
    _conv_shift(xpad, csdA, csdB, H, W, Cin)
    yield
    acc = _conv_dots(csdA, csdB, wdc_ref, H, W, Cin) + bdc_ref[...]
    yield
    for ph in range(2):
        x0[ph:Hx:2, :, :] = acc[:, 2 * ph * C:2 * (ph + 1) * C].reshape(
            H, Wx, C)
    yield
    _zero_halo(segpad, _F32)
    sv = seg_ref[i].astype(_F32)
    for a in range(2):
        for b in range(2):
            segpad[1 + a:Hx + 1:2, 1 + b:Wx + 1:2, :] = sv
    yield
    _conv_shift(segpad, cssA, cssB, Hx, Wx, Cs)
    yield
    mlp = _conv_dots(cssA, cssB, wsh_ref, Hx, Wx, Cs) + bsh_ref[...]
    mlp = jnp.maximum(mlp, 0.0)
    yield
    _zero_halo(actpad0, _BF16)
    _zero_halo(actpad1, _BF16)
    actpad0[1:Hx + 1, 1:Wx + 1, :] = (
        mlp[:, :nh].astype(_BF16).reshape(Hx, Wx, nh))
    actpad1[1:Hx + 1, 1:Wx + 1, :] = (
        mlp[:, nh:].astype(_BF16).reshape(Hx, Wx, nh))
    _zero_halo(spad0, _BF16)
    _zero_halo(spad1, _BF16)
    yield
    x0f = x0[...].reshape(HW, C)

    def modulate(h2d, gb):
        inv, nbias = _norm_scale_bias(h2d)
        nrm = h2d * inv[0] + nbias[0]
        s = nrm * (1.0 + gb[:, :C]) + gb[:, C:]
        s = jnp.where(s >= 0, s, 0.2 * s)
        return s.astype(_BF16)

    _conv_shift(actpad0, csgA, csgB, Hx, Wx, nh)
    yield
    gb = _conv_dots(csgA, csgB, wgb_ref[0], Hx, Wx, nh) + bgb_ref[0]
    yield
    spad0[1:Hx + 1, 1:Wx + 1, :] = modulate(x0f, gb).reshape(Hx, Wx, C)
    yield
    _conv_shift(spad0, cscA, cscB, Hx, Wx, C)
    yield
    dx = _conv_dots(cscA, cscB, wcv_ref[0], Hx, Wx, C) + bcv_ref[0]
    yield
    _conv_shift(actpad1, csgA, csgB, Hx, Wx, nh)
    yield
    gb = _conv_dots(csgA, csgB, wgb_ref[1], Hx, Wx, nh) + bgb_ref[1]
    yield
    spad1[1:Hx + 1, 1:Wx + 1, :] = modulate(dx, gb).reshape(Hx, Wx, C)
    yield
    _conv_shift(spad1, cscA, cscB, Hx, Wx, C)
    yield
    dx = _conv_dots(cscA, cscB, wcv_ref[1], Hx, Wx, C) + bcv_ref[1]
    yield
    y = _instance_norm_rows(x0f + dx)
    yield
    o_ref[i] = y.T


_SPP = 3
_SKEW = 2


def _fused_kernel(x_ref, seg_ref, wdc_ref, bdc_ref, wsh_ref, bsh_ref,
                  wgb_ref, bgb_ref, wcv_ref, bcv_ref, o_ref, *scr,
                  H, W, Cin, C, Cs, nh):
    nper = len(scr) // _SPP
    gens = [_sample_stages(i, x_ref, seg_ref, wdc_ref, bdc_ref, wsh_ref,
                           bsh_ref, wgb_ref, bgb_ref, wcv_ref, bcv_ref,
                           o_ref, scr[i * nper:(i + 1) * nper],
                           H, W, Cin, C, Cs, nh)
            for i in range(_SPP)]
    live = [True] * _SPP
    for j in range(_SPP):
        for _ in range((_SPP - 1 - j) * _SKEW):
            next(gens[j], None)
    while any(live):
        for i in range(_SPP):
            if live[i] and next(gens[i], "done") == "done":
                live[i] = False


def kernel(inp, hsv, w_deconv, b_deconv, wsh, bsh, wgb, bgb, wcv, bcv):
    N, H, W, Cin = inp.shape
    Cs = hsv.shape[-1]
    nh = wsh.shape[-1]
    C = wcv.shape[-1]
    assert W % 8 == 0 and hsv.shape[1] == H and hsv.shape[2] == W
    assert Cs == nh == C and N % _SPP == 0
    Hx, Wx = 2 * H, 2 * W
    HW = Hx * Wx

    x_bf = inp.astype(_BF16)
    seg_bf = hsv.astype(_BF16)
    wdc = _perm_taps(w_deconv, Cin).astype(_BF16)
    b4 = jnp.tile(b_deconv.astype(_F32), 4).reshape(1, 4 * C)
    wsh_c = _perm_taps(jnp.concatenate([wsh[0], wsh[1]], axis=-1),
                       Cs).astype(_BF16)
    bsh_c = jnp.concatenate([bsh[0], bsh[1]], axis=-1).astype(_F32)
    wgb_c = jnp.stack([_perm_taps(wgb.reshape(2, 9 * nh, 2 * C)[i], nh)
                       for i in range(2)]).astype(_BF16)
    wcv_c = jnp.stack([_perm_taps(wcv[i], C)
                       for i in range(2)]).astype(_BF16)
    bgb_c = bgb.astype(_F32)
    bcv_c = bcv.astype(_F32)

    flops = 2 * N * (H * W * 9 * Cin * 4 * C + HW * 9 * Cs * 2 * nh
                     + 2 * HW * (9 * nh * 2 * C + 9 * C * C))
    bytes_accessed = 2 * (N * H * W * (Cin + Cs) + 9 * Cin * 4 * C
                          + 9 * Cs * 2 * nh + 2 * 9 * nh * 2 * C
                          + 2 * 9 * C * C) + 4 * N * HW * C

    out = pl.pallas_call(
        functools.partial(_fused_kernel, H=H, W=W, Cin=Cin, C=C, Cs=Cs,
                          nh=nh),
        out_shape=jax.ShapeDtypeStruct((N, C, HW), _F32),
        grid=(N // _SPP,),
        in_specs=[
            pl.BlockSpec((_SPP, H, W, Cin), lambda n: (n, 0, 0, 0)),
            pl.BlockSpec((_SPP, H, W, Cs), lambda n: (n, 0, 0, 0)),
            pl.BlockSpec((9 * Cin, 4 * C), lambda n: (0, 0)),
            pl.BlockSpec((1, 4 * C), lambda n: (0, 0)),
            pl.BlockSpec((9 * Cs, 2 * nh), lambda n: (0, 0)),
            pl.BlockSpec((1, 2 * nh), lambda n: (0, 0)),
            pl.BlockSpec((2, 9 * nh, 2 * C), lambda n: (0, 0, 0)),
            pl.BlockSpec((2, 1, 2 * C), lambda n: (0, 0, 0)),
            pl.BlockSpec((2, 9 * C, C), lambda n: (0, 0, 0)),
            pl.BlockSpec((2, 1, C), lambda n: (0, 0, 0)),
        ],
        out_specs=pl.BlockSpec((_SPP, C, HW), lambda n: (n, 0, 0)),
        scratch_shapes=[
            s for _ in range(_SPP) for s in [
                pltpu.VMEM((H + 2, W + 2, Cin), _BF16),
                pltpu.VMEM((H + 2, W, 2 * Cin), _BF16),
                pltpu.VMEM((H + 2, W, 2 * Cin), _BF16),
                pltpu.VMEM((Hx, Wx, C), _F32),
                pltpu.VMEM((Hx + 2, Wx + 2, Cs), _F32),
                pltpu.VMEM((Hx + 2, Wx, 2 * C), _BF16),
                pltpu.VMEM((Hx + 2, Wx, 2 * C), _BF16),
                pltpu.VMEM((Hx + 2, Wx, 2 * C), _BF16),
                pltpu.VMEM((Hx + 2, Wx, 2 * C), _BF16),
                pltpu.VMEM((Hx + 2, Wx, 2 * C), _BF16),
                pltpu.VMEM((Hx + 2, Wx, 2 * C), _BF16),
                pltpu.VMEM((Hx + 2, Wx + 2, nh), _BF16),
                pltpu.VMEM((Hx + 2, Wx + 2, nh), _BF16),
                pltpu.VMEM((Hx + 2, Wx + 2, C), _BF16),
                pltpu.VMEM((Hx + 2, Wx + 2, C), _BF16),
            ]],
        compiler_params=pltpu.CompilerParams(
            dimension_semantics=("parallel",),
            vmem_limit_bytes=_vmem_limit()),
        cost_estimate=pl.CostEstimate(flops=flops, transcendentals=0,
                                      bytes_accessed=bytes_accessed),
    )(x_bf, seg_bf, wdc, b4, wsh_c, bsh_c, wgb_c, bgb_c, wcv_c, bcv_c)
    return out.reshape(N, C, Hx, Wx)


# speedup vs baseline: 1.9555x; 1.0404x over previous
import functools

import jax
import jax.numpy as jnp
from jax.experimental import pallas as pl
from jax.experimental.pallas import tpu as pltpu

_EPS = 1e-5
_F32 = jnp.float32
_BF16 = jnp.bfloat16

_TAP_PERM = (0, 1, 3, 4, 6, 7, 2, 5, 8)


def _vmem_limit():
    cap = 64 * 1024 * 1024
    return int(min((cap * 3) // 4, 100 * 1024 * 1024))


def _perm_taps(w9c, c):
    n = w9c.shape[-1]
    return w9c.reshape(9, c, n)[jnp.array(_TAP_PERM)].reshape(9 * c, n)


def _zero_halo(ref, dtype):
    hp, wp, c = ref.shape
    ref[0:1, :, :] = jnp.zeros((1, wp, c), dtype)
    ref[hp - 1:hp, :, :] = jnp.zeros((1, wp, c), dtype)
    ref[:, 0:1, :] = jnp.zeros((hp, 1, c), dtype)
    ref[:, wp - 1:wp, :] = jnp.zeros((hp, 1, c), dtype)


def _instance_norm_rows(x2d, eps=_EPS):
    mean = jnp.mean(x2d, axis=0, keepdims=True)
    msq = jnp.mean(x2d * x2d, axis=0, keepdims=True)
    var = msq - mean * mean
    return (x2d - mean) * jax.lax.rsqrt(var + eps)


def _norm_scale_bias(x2d, eps=_EPS):
    mean = jnp.mean(x2d, axis=0, keepdims=True)
    msq = jnp.mean(x2d * x2d, axis=0, keepdims=True)
    inv = jax.lax.rsqrt(msq - mean * mean + eps)
    return inv, -mean * inv


def _conv_shift(pad_ref, csA, csB, h, w, c, c0=0):
    c1 = c0 + c
    csA[:, :, 0:c] = pad_ref[:, 0:w, c0:c1].astype(_BF16)
    csA[:, :, c:2 * c] = pad_ref[:, 1:w + 1, c0:c1].astype(_BF16)
    csB[:, :, 0:c] = pad_ref[:, 2:w + 2, c0:c1].astype(_BF16)
    csB[0:h + 1, :, c:2 * c] = csB[1:h + 2, :, 0:c]


def _conv_dots(csA, csB, w_ref, h, w, c):
    a = csA[...].reshape((h + 2) * w, 2 * c)
    bb = csB[...].reshape((h + 2) * w, 2 * c)
    hw = h * w
    dot = functools.partial(jnp.dot, preferred_element_type=jnp.float32)
    acc = dot(a[0:hw], w_ref[0:2 * c])
    acc = acc + dot(a[w:w + hw], w_ref[2 * c:4 * c])
    acc = acc + dot(a[2 * w:2 * w + hw], w_ref[4 * c:6 * c])
    acc = acc + dot(bb[0:hw], w_ref[6 * c:8 * c])
    acc = acc + dot(bb[2 * w:2 * w + hw, 0:c], w_ref[8 * c:9 * c])
    return acc


def _sample_stages(i, x_ref, seg_ref, wdc_ref, bdc_ref, wsh_ref, bsh_ref,
                   wgb_ref, bgb_ref, wcv_ref, bcv_ref, o_ref, scr,
                   H, W, Cin, C, Cs, nh):
    (xpad, csdA, csdB, x0, segpad, cssA, cssB, csgA, csgB, cscA, cscB,
     actpad01, spad0, spad1) = scr
    Hx, Wx = 2 * H, 2 * W
    HW = Hx * Wx

    _zero_halo(xpad, _BF16)
    xpad[1:H + 1, 1:W + 1, :] = jnp.maximum(x_ref[i], 0)
    _conv_shift(xpad, csdA, csdB, H, W, Cin)
    yield
    ad = csdA[...].reshape((H + 2) * W, 2 * Cin)
    bd = csdB[...].reshape((H + 2) * W, 2 * Cin)
    dot = functools.partial(jnp.dot, preferred_element_type=jnp.float32)
    for ph in range(2):
        acc = dot(ad[ph * W:ph * W + H * W], wdc_ref[ph, 0])
        acc = acc + dot(ad[(ph + 1) * W:(ph + 1) * W + H * W], wdc_ref[ph, 1])
        acc = acc + dot(bd[ph * W:ph * W + H * W], wdc_ref[ph, 2])
        acc = acc + bdc_ref[...]
        x0[ph:Hx:2, :, :] = acc.reshape(H, Wx, C)
        yield
    _zero_halo(segpad, _F32)
    sv = seg_ref[i].astype(_F32)
    for a in range(2):
        for b in range(2):
            segpad[1 + a:Hx + 1:2, 1 + b:Wx + 1:2, :] = sv
    yield
    _conv_shift(segpad, cssA, cssB, Hx, Wx, Cs)
    yield
    mlp = _conv_dots(cssA, cssB, wsh_ref, Hx, Wx, Cs) + bsh_ref[...]
    yield
    _zero_halo(actpad01, _BF16)
    mlpb = jnp.maximum(mlp.astype(_BF16), 0)
    actpad01[1:Hx + 1, 1:Wx + 1, :] = mlpb.reshape(Hx, Wx, 2 * nh)
    _zero_halo(spad0, _BF16)
    _zero_halo(spad1, _BF16)
    yield
    x0f = x0[...].reshape(HW, C)

    def modulate(h2d, gb):
        inv, nbias = _norm_scale_bias(h2d)
        nrm = h2d * inv[0] + nbias[0]
        s = nrm * (1.0 + gb[:, :C]) + gb[:, C:]
        s = jnp.where(s >= 0, s, 0.2 * s)
        return s.astype(_BF16)

    _conv_shift(actpad01, csgA, csgB, Hx, Wx, nh, 0)
    yield
    gb = _conv_dots(csgA, csgB, wgb_ref[0], Hx, Wx, nh) + bgb_ref[0]
    yield
    spad0[1:Hx + 1, 1:Wx + 1, :] = modulate(x0f, gb).reshape(Hx, Wx, C)
    yield
    _conv_shift(spad0, cscA, cscB, Hx, Wx, C)
    yield
    dx = _conv_dots(cscA, cscB, wcv_ref[0], Hx, Wx, C) + bcv_ref[0]
    yield
    _conv_shift(actpad01, csgA, csgB, Hx, Wx, nh, nh)
    yield
    gb = _conv_dots(csgA, csgB, wgb_ref[1], Hx, Wx, nh) + bgb_ref[1]
    yield
    spad1[1:Hx + 1, 1:Wx + 1, :] = modulate(dx, gb).reshape(Hx, Wx, C)
    yield
    _conv_shift(spad1, cscA, cscB, Hx, Wx, C)
    yield
    dx = _conv_dots(cscA, cscB, wcv_ref[1], Hx, Wx, C) + bcv_ref[1]
    yield
    y = _instance_norm_rows(x0f + dx)
    yield
    o_ref[i] = y.T


_SPP = 3
_SKEW = 2


def _fused_kernel(x_ref, seg_ref, wdc_ref, bdc_ref, wsh_ref, bsh_ref,
                  wgb_ref, bgb_ref, wcv_ref, bcv_ref, o_ref, *scr,
                  H, W, Cin, C, Cs, nh):
    nper = len(scr) // _SPP
    gens = [_sample_stages(i, x_ref, seg_ref, wdc_ref, bdc_ref, wsh_ref,
                           bsh_ref, wgb_ref, bgb_ref, wcv_ref, bcv_ref,
                           o_ref, scr[i * nper:(i + 1) * nper],
                           H, W, Cin, C, Cs, nh)
            for i in range(_SPP)]
    live = [True] * _SPP
    for j in range(_SPP):
        for _ in range((_SPP - 1 - j) * _SKEW):
            next(gens[j], None)
    while any(live):
        for i in range(_SPP):
            if live[i] and next(gens[i], "done") == "done":
                live[i] = False


def kernel(inp, hsv, w_deconv, b_deconv, wsh, bsh, wgb, bgb, wcv, bcv):
    N, H, W, Cin = inp.shape
    Cs = hsv.shape[-1]
    nh = wsh.shape[-1]
    C = wcv.shape[-1]
    assert W % 8 == 0 and hsv.shape[1] == H and hsv.shape[2] == W
    assert Cs == nh == C and N % _SPP == 0
    Hx, Wx = 2 * H, 2 * W
    HW = Hx * Wx

    x_bf = inp.astype(_BF16)
    seg_bf = hsv.astype(_BF16)
    t9 = w_deconv.reshape(3, 3, Cin, 4 * C)
    wdc = jnp.stack([
        jnp.stack([
            jnp.concatenate([t9[u, 0, :, cols], t9[u, 1, :, cols]], axis=0)
            if k < 2 else
            jnp.concatenate([t9[ph, 2, :, cols], t9[ph + 1, 2, :, cols]],
                            axis=0)
            for k, u in ((0, ph), (1, ph + 1), (2, ph))])
        for ph, cols in ((0, slice(0, 2 * C)), (1, slice(2 * C, 4 * C)))
    ]).astype(_BF16)
    b4 = jnp.tile(b_deconv.astype(_F32), 2).reshape(1, 2 * C)
    wsh_c = _perm_taps(jnp.concatenate([wsh[0], wsh[1]], axis=-1),
                       Cs).astype(_BF16)
    bsh_c = jnp.concatenate([bsh[0], bsh[1]], axis=-1).astype(_F32)
    wgb_c = jnp.stack([_perm_taps(wgb.reshape(2, 9 * nh, 2 * C)[i], nh)
                       for i in range(2)]).astype(_BF16)
    wcv_c = jnp.stack([_perm_taps(wcv[i], C)
                       for i in range(2)]).astype(_BF16)
    bgb_c = bgb.astype(_F32)
    bcv_c = bcv.astype(_F32)

    flops = 2 * N * (H * W * 9 * Cin * 4 * C + HW * 9 * Cs * 2 * nh
                     + 2 * HW * (9 * nh * 2 * C + 9 * C * C))
    bytes_accessed = 2 * (N * H * W * (Cin + Cs) + 9 * Cin * 4 * C
                          + 9 * Cs * 2 * nh + 2 * 9 * nh * 2 * C
                          + 2 * 9 * C * C) + 4 * N * HW * C

    out = pl.pallas_call(
        functools.partial(_fused_kernel, H=H, W=W, Cin=Cin, C=C, Cs=Cs,
                          nh=nh),
        out_shape=jax.ShapeDtypeStruct((N, C, HW), _F32),
        grid=(N // _SPP,),
        in_specs=[
            pl.BlockSpec((_SPP, H, W, Cin), lambda n: (n, 0, 0, 0)),
            pl.BlockSpec((_SPP, H, W, Cs), lambda n: (n, 0, 0, 0)),
            pl.BlockSpec((2, 3, 2 * Cin, 2 * C), lambda n: (0, 0, 0, 0)),
            pl.BlockSpec((1, 2 * C), lambda n: (0, 0)),
            pl.BlockSpec((9 * Cs, 2 * nh), lambda n: (0, 0)),
            pl.BlockSpec((1, 2 * nh), lambda n: (0, 0)),
            pl.BlockSpec((2, 9 * nh, 2 * C), lambda n: (0, 0, 0)),
            pl.BlockSpec((2, 1, 2 * C), lambda n: (0, 0, 0)),
            pl.BlockSpec((2, 9 * C, C), lambda n: (0, 0, 0)),
            pl.BlockSpec((2, 1, C), lambda n: (0, 0, 0)),
        ],
        out_specs=pl.BlockSpec((_SPP, C, HW), lambda n: (n, 0, 0)),
        scratch_shapes=[
            s for _ in range(_SPP) for s in [
                pltpu.VMEM((H + 2, W + 2, Cin), _BF16),
                pltpu.VMEM((H + 2, W, 2 * Cin), _BF16),
                pltpu.VMEM((H + 2, W, 2 * Cin), _BF16),
                pltpu.VMEM((Hx, Wx, C), _F32),
                pltpu.VMEM((Hx + 2, Wx + 2, Cs), _F32),
                pltpu.VMEM((Hx + 2, Wx, 2 * C), _BF16),
                pltpu.VMEM((Hx + 2, Wx, 2 * C), _BF16),
                pltpu.VMEM((Hx + 2, Wx, 2 * C), _BF16),
                pltpu.VMEM((Hx + 2, Wx, 2 * C), _BF16),
                pltpu.VMEM((Hx + 2, Wx, 2 * C), _BF16),
                pltpu.VMEM((Hx + 2, Wx, 2 * C), _BF16),
                pltpu.VMEM((Hx + 2, Wx + 2, 2 * nh), _BF16),
                pltpu.VMEM((Hx + 2, Wx + 2, C), _BF16),
                pltpu.VMEM((Hx + 2, Wx + 2, C), _BF16),
            ]],
        compiler_params=pltpu.CompilerParams(
            dimension_semantics=("parallel",),
            vmem_limit_bytes=_vmem_limit()),
        cost_estimate=pl.CostEstimate(flops=flops, transcendentals=0,
                                      bytes_accessed=bytes_accessed),
    )(x_bf, seg_bf, wdc, b4, wsh_c, bsh_c, wgb_c, bgb_c, wcv_c, bcv_c)
    return out.reshape(N, C, Hx, Wx)


# speedup vs baseline: 2.0909x; 1.0692x over previous
import functools

import jax
import jax.numpy as jnp
from jax.experimental import pallas as pl
from jax.experimental.pallas import tpu as pltpu

_EPS = 1e-5
_F32 = jnp.float32
_BF16 = jnp.bfloat16

_TAP_PERM = (0, 1, 3, 4, 6, 7, 2, 5, 8)


def _vmem_limit():
    cap = 64 * 1024 * 1024
    return int(min((cap * 3) // 4, 100 * 1024 * 1024))


def _perm_taps(w9c, c):
    n = w9c.shape[-1]
    return w9c.reshape(9, c, n)[jnp.array(_TAP_PERM)].reshape(9 * c, n)


def _zero_halo(ref, dtype):
    hp, wp, c = ref.shape
    ref[0:1, :, :] = jnp.zeros((1, wp, c), dtype)
    ref[hp - 1:hp, :, :] = jnp.zeros((1, wp, c), dtype)
    ref[:, 0:1, :] = jnp.zeros((hp, 1, c), dtype)
    ref[:, wp - 1:wp, :] = jnp.zeros((hp, 1, c), dtype)


def _instance_norm_rows(x2d, eps=_EPS):
    mean = jnp.mean(x2d, axis=0, keepdims=True)
    msq = jnp.mean(x2d * x2d, axis=0, keepdims=True)
    var = msq - mean * mean
    return (x2d - mean) * jax.lax.rsqrt(var + eps)


def _norm_scale_bias(x2d, eps=_EPS):
    mean = jnp.mean(x2d, axis=0, keepdims=True)
    msq = jnp.mean(x2d * x2d, axis=0, keepdims=True)
    inv = jax.lax.rsqrt(msq - mean * mean + eps)
    return inv, -mean * inv


def _conv_shift(pad_ref, csA, csB, h, w, c, c0=0):
    c1 = c0 + c
    csA[:, :, 0:c] = pad_ref[:, 0:w, c0:c1].astype(_BF16)
    csA[:, :, c:2 * c] = pad_ref[:, 1:w + 1, c0:c1].astype(_BF16)
    csB[:, :, 0:c] = pad_ref[:, 2:w + 2, c0:c1].astype(_BF16)
    csB[0:h + 1, :, c:2 * c] = csB[1:h + 2, :, 0:c]


def _cs_fill(v, csA, csB, h, w, c):
    z = jnp.zeros((1, w, c), _BF16)
    for cs in (csA, csB):
        cs[0:1, :, 0:c] = z
        cs[0:1, :, c:2 * c] = z
        cs[h + 1:h + 2, :, 0:c] = z
        cs[h + 1:h + 2, :, c:2 * c] = z
    csA[1:h + 1, :, c:2 * c] = v
    csA[1:h + 1, 0:1, 0:c] = jnp.zeros((h, 1, c), _BF16)
    csA[1:h + 1, 1:w, 0:c] = v[:, 0:w - 1, :]
    csB[1:h + 1, w - 1:w, 0:c] = jnp.zeros((h, 1, c), _BF16)
    csB[1:h + 1, 0:w - 1, 0:c] = v[:, 1:w, :]
    csB[0:h + 1, :, c:2 * c] = csB[1:h + 2, :, 0:c]


def _conv_dots(csA, csB, w_ref, h, w, c):
    a = csA[...].reshape((h + 2) * w, 2 * c)
    bb = csB[...].reshape((h + 2) * w, 2 * c)
    hw = h * w
    dot = functools.partial(jnp.dot, preferred_element_type=jnp.float32)
    acc = dot(a[0:hw], w_ref[0:2 * c])
    acc = acc + dot(a[w:w + hw], w_ref[2 * c:4 * c])
    acc = acc + dot(a[2 * w:2 * w + hw], w_ref[4 * c:6 * c])
    acc = acc + dot(bb[0:hw], w_ref[6 * c:8 * c])
    acc = acc + dot(bb[2 * w:2 * w + hw, 0:c], w_ref[8 * c:9 * c])
    return acc


def _sample_stages(i, x_ref, seg_ref, wdc_ref, bdc_ref, wsh_ref, bsh_ref,
                   wgb_ref, bgb_ref, wcv_ref, bcv_ref, o_ref, scr,
                   H, W, Cin, C, Cs, nh):
    (csdA, csdB, x0, segpad, cssA, cssB, csgA, csgB, cscA, cscB,
     actpad01) = scr
    Hx, Wx = 2 * H, 2 * W
    HW = Hx * Wx

    _cs_fill(jnp.maximum(x_ref[i], 0), csdA, csdB, H, W, Cin)
    yield
    ad = csdA[...].reshape((H + 2) * W, 2 * Cin)
    bd = csdB[...].reshape((H + 2) * W, 2 * Cin)
    dot = functools.partial(jnp.dot, preferred_element_type=jnp.float32)
    for ph in range(2):
        acc = dot(ad[ph * W:ph * W + H * W], wdc_ref[ph, 0])
        acc = acc + dot(ad[(ph + 1) * W:(ph + 1) * W + H * W], wdc_ref[ph, 1])
        acc = acc + dot(bd[ph * W:ph * W + H * W], wdc_ref[ph, 2])
        acc = acc + bdc_ref[...]
        x0[ph:Hx:2, :, :] = acc.reshape(H, Wx, C)
        yield
    _zero_halo(segpad, _F32)
    sv = seg_ref[i].astype(_F32)
    for a in range(2):
        for b in range(2):
            segpad[1 + a:Hx + 1:2, 1 + b:Wx + 1:2, :] = sv
    yield
    _conv_shift(segpad, cssA, cssB, Hx, Wx, Cs)
    yield
    mlp = _conv_dots(cssA, cssB, wsh_ref, Hx, Wx, Cs) + bsh_ref[...]
    yield
    _zero_halo(actpad01, _BF16)
    mlpb = jnp.maximum(mlp.astype(_BF16), 0)
    actpad01[1:Hx + 1, 1:Wx + 1, :] = mlpb.reshape(Hx, Wx, 2 * nh)
    yield
    x0f = x0[...].reshape(HW, C)

    def modulate(h2d, gb):
        inv, nbias = _norm_scale_bias(h2d)
        nrm = h2d * inv[0] + nbias[0]
        s = nrm * (1.0 + gb[:, :C]) + gb[:, C:]
        s = jnp.where(s >= 0, s, 0.2 * s)
        return s.astype(_BF16)

    _conv_shift(actpad01, csgA, csgB, Hx, Wx, nh, 0)
    yield
    gb = _conv_dots(csgA, csgB, wgb_ref[0], Hx, Wx, nh) + bgb_ref[0]
    yield
    _cs_fill(modulate(x0f, gb).reshape(Hx, Wx, C), cscA, cscB, Hx, Wx, C)
    yield
    dx = _conv_dots(cscA, cscB, wcv_ref[0], Hx, Wx, C) + bcv_ref[0]
    yield
    _conv_shift(actpad01, csgA, csgB, Hx, Wx, nh, nh)
    yield
    gb = _conv_dots(csgA, csgB, wgb_ref[1], Hx, Wx, nh) + bgb_ref[1]
    yield
    _cs_fill(modulate(dx, gb).reshape(Hx, Wx, C), cscA, cscB, Hx, Wx, C)
    yield
    dx = _conv_dots(cscA, cscB, wcv_ref[1], Hx, Wx, C) + bcv_ref[1]
    yield
    y = _instance_norm_rows(x0f + dx)
    yield
    o_ref[i] = y.T


_SPP = 3
_SKEW = 3


def _fused_kernel(x_ref, seg_ref, wdc_ref, bdc_ref, wsh_ref, bsh_ref,
                  wgb_ref, bgb_ref, wcv_ref, bcv_ref, o_ref, *scr,
                  H, W, Cin, C, Cs, nh):
    nper = len(scr) // _SPP
    gens = [_sample_stages(i, x_ref, seg_ref, wdc_ref, bdc_ref, wsh_ref,
                           bsh_ref, wgb_ref, bgb_ref, wcv_ref, bcv_ref,
                           o_ref, scr[i * nper:(i + 1) * nper],
                           H, W, Cin, C, Cs, nh)
            for i in range(_SPP)]
    live = [True] * _SPP
    for j in range(_SPP):
        for _ in range((_SPP - 1 - j) * _SKEW):
            next(gens[j], None)
    while any(live):
        for i in range(_SPP):
            if live[i] and next(gens[i], "done") == "done":
                live[i] = False


def kernel(inp, hsv, w_deconv, b_deconv, wsh, bsh, wgb, bgb, wcv, bcv):
    N, H, W, Cin = inp.shape
    Cs = hsv.shape[-1]
    nh = wsh.shape[-1]
    C = wcv.shape[-1]
    assert W % 8 == 0 and hsv.shape[1] == H and hsv.shape[2] == W
    assert Cs == nh == C and N % _SPP == 0
    Hx, Wx = 2 * H, 2 * W
    HW = Hx * Wx

    x_bf = inp.astype(_BF16)
    seg_bf = hsv.astype(_BF16)
    t9 = w_deconv.reshape(3, 3, Cin, 4 * C)
    wdc = jnp.stack([
        jnp.stack([
            jnp.concatenate([t9[u, 0, :, cols], t9[u, 1, :, cols]], axis=0)
            if k < 2 else
            jnp.concatenate([t9[ph, 2, :, cols], t9[ph + 1, 2, :, cols]],
                            axis=0)
            for k, u in ((0, ph), (1, ph + 1), (2, ph))])
        for ph, cols in ((0, slice(0, 2 * C)), (1, slice(2 * C, 4 * C)))
    ]).astype(_BF16)
    b4 = jnp.tile(b_deconv.astype(_F32), 2).reshape(1, 2 * C)
    wsh_c = _perm_taps(jnp.concatenate([wsh[0], wsh[1]], axis=-1),
                       Cs).astype(_BF16)
    bsh_c = jnp.concatenate([bsh[0], bsh[1]], axis=-1).astype(_F32)
    wgb_c = jnp.stack([_perm_taps(wgb.reshape(2, 9 * nh, 2 * C)[i], nh)
                       for i in range(2)]).astype(_BF16)
    wcv_c = jnp.stack([_perm_taps(wcv[i], C)
                       for i in range(2)]).astype(_BF16)
    bgb_c = bgb.astype(_F32)
    bcv_c = bcv.astype(_F32)

    flops = 2 * N * (H * W * 9 * Cin * 4 * C + HW * 9 * Cs * 2 * nh
                     + 2 * HW * (9 * nh * 2 * C + 9 * C * C))
    bytes_accessed = 2 * (N * H * W * (Cin + Cs) + 9 * Cin * 4 * C
                          + 9 * Cs * 2 * nh + 2 * 9 * nh * 2 * C
                          + 2 * 9 * C * C) + 4 * N * HW * C

    out = pl.pallas_call(
        functools.partial(_fused_kernel, H=H, W=W, Cin=Cin, C=C, Cs=Cs,
                          nh=nh),
        out_shape=jax.ShapeDtypeStruct((N, C, HW), _F32),
        grid=(N // _SPP,),
        in_specs=[
            pl.BlockSpec((_SPP, H, W, Cin), lambda n: (n, 0, 0, 0)),
            pl.BlockSpec((_SPP, H, W, Cs), lambda n: (n, 0, 0, 0)),
            pl.BlockSpec((2, 3, 2 * Cin, 2 * C), lambda n: (0, 0, 0, 0)),
            pl.BlockSpec((1, 2 * C), lambda n: (0, 0)),
            pl.BlockSpec((9 * Cs, 2 * nh), lambda n: (0, 0)),
            pl.BlockSpec((1, 2 * nh), lambda n: (0, 0)),
            pl.BlockSpec((2, 9 * nh, 2 * C), lambda n: (0, 0, 0)),
            pl.BlockSpec((2, 1, 2 * C), lambda n: (0, 0, 0)),
            pl.BlockSpec((2, 9 * C, C), lambda n: (0, 0, 0)),
            pl.BlockSpec((2, 1, C), lambda n: (0, 0, 0)),
        ],
        out_specs=pl.BlockSpec((_SPP, C, HW), lambda n: (n, 0, 0)),
        scratch_shapes=[
            s for _ in range(_SPP) for s in [
                pltpu.VMEM((H + 2, W, 2 * Cin), _BF16),
                pltpu.VMEM((H + 2, W, 2 * Cin), _BF16),
                pltpu.VMEM((Hx, Wx, C), _F32),
                pltpu.VMEM((Hx + 2, Wx + 2, Cs), _F32),
                pltpu.VMEM((Hx + 2, Wx, 2 * C), _BF16),
                pltpu.VMEM((Hx + 2, Wx, 2 * C), _BF16),
                pltpu.VMEM((Hx + 2, Wx, 2 * C), _BF16),
                pltpu.VMEM((Hx + 2, Wx, 2 * C), _BF16),
                pltpu.VMEM((Hx + 2, Wx, 2 * C), _BF16),
                pltpu.VMEM((Hx + 2, Wx, 2 * C), _BF16),
                pltpu.VMEM((Hx + 2, Wx + 2, 2 * nh), _BF16),
            ]],
        compiler_params=pltpu.CompilerParams(
            dimension_semantics=("parallel",),
            vmem_limit_bytes=_vmem_limit()),
        cost_estimate=pl.CostEstimate(flops=flops, transcendentals=0,
                                      bytes_accessed=bytes_accessed),
    )(x_bf, seg_bf, wdc, b4, wsh_c, bsh_c, wgb_c, bgb_c, wcv_c, bcv_c)
    return out.reshape(N, C, Hx, Wx)
